# baseline (speedup 1.0000x reference)
import functools

import jax
import jax.numpy as jnp
from jax import lax
from jax.experimental import pallas as pl
from jax.experimental.pallas import tpu as pltpu
from jax.experimental.pallas import tpu_sc as plsc

_D = 128
_CHUNK = 128
_GPS = 1
_SUP = _CHUNK * _GPS
_NBUF = 7
_DIST = 5


def _embedding_lookup(idx_flat, weight, *, num_workers, b_per_w):
    mesh = plsc.VectorSubcoreMesh(core_axis_name="c", subcore_axis_name="s")
    n_chunks = b_per_w // _CHUNK
    n_sup = b_per_w // _SUP
    n_rounds = (n_sup + _NBUF - 1) // _NBUF

    @functools.partial(
        pl.kernel,
        mesh=mesh,
        out_type=jax.ShapeDtypeStruct((idx_flat.shape[0], _D), jnp.float32),
        scratch_types=[
            pltpu.VMEM((n_chunks, _CHUNK), jnp.int32),
            pltpu.VMEM((_NBUF, _SUP, _D), jnp.float32),
            pltpu.SemaphoreType.DMA((_NBUF,)),
            pltpu.SemaphoreType.DMA((_NBUF,)),
        ],
    )
    def k(idx_hbm, table_hbm, out_hbm, idx_v, rows_v, gsem, osem):
        num_cores = jax.lax.axis_size("c")
        wid = lax.axis_index("s") * num_cores + lax.axis_index("c")
        base = wid * b_per_w

        pltpu.sync_copy(idx_hbm.at[wid], idx_v)

        def gather_start(t, b):
            for g in range(_GPS):
                pltpu.async_copy(
                    table_hbm.at[idx_v.at[t * _GPS + g]],
                    rows_v.at[b].at[pl.ds(g * _CHUNK, _CHUNK)],
                    gsem.at[b],
                )

        def gather_wait(b):
            for g in range(_GPS):
                pltpu.make_async_copy(
                    table_hbm.at[idx_v.at[0]],
                    rows_v.at[b].at[pl.ds(g * _CHUNK, _CHUNK)],
                    gsem.at[b],
                ).wait()

        def out_start(t, b):
            pltpu.async_copy(
                rows_v.at[b], out_hbm.at[pl.ds(base + t * _SUP, _SUP)],
                osem.at[b],
            )

        def out_wait(b):
            pltpu.make_async_copy(
                rows_v.at[b], out_hbm.at[pl.ds(base, _SUP)], osem.at[b]
            ).wait()

        for t in range(_DIST):
            gather_start(t, t % _NBUF)

        def body(i, carry):
            for b in range(_NBUF):
                t = i * _NBUF + b
                bg = (b + _DIST) % _NBUF
                tg = t + _DIST

                @pl.when(jnp.logical_and(tg >= _NBUF, tg < n_sup))
                def _():
                    out_wait(bg)

                @pl.when(tg < n_sup)
                def _():
                    gather_start(tg, bg)

                @pl.when(t < n_sup)
                def _():
                    gather_wait(b)
                    out_start(t, b)

            return carry

        lax.fori_loop(0, n_rounds, body, 0)

        for b in range(_NBUF):
            out_wait(b)

    idx3 = idx_flat.reshape(num_workers, n_chunks, _CHUNK)
    return k(idx3, weight)


def kernel(token_ids, weight):
    n, t = token_ids.shape
    idx_flat = token_ids.T.reshape(-1).astype(jnp.int32)
    b = idx_flat.shape[0]
    num_workers = 32
    assert b % (num_workers * _SUP) == 0
    out = _embedding_lookup(
        idx_flat, weight, num_workers=num_workers, b_per_w=b // num_workers
    )
    return out.reshape(t, n, _D).transpose(1, 0, 2)


# speedup vs baseline: 1.0003x; 1.0003x over previous
import functools

import jax
import jax.numpy as jnp
from jax import lax
from jax.experimental import pallas as pl
from jax.experimental.pallas import tpu as pltpu
from jax.experimental.pallas import tpu_sc as plsc

_D = 128
_CHUNK = 128
_GPS = 1
_SUP = _CHUNK * _GPS
_NBUF = 7
_DIST = 4


def _embedding_lookup(idx_flat, weight, *, num_workers, b_per_w):
    mesh = plsc.VectorSubcoreMesh(core_axis_name="c", subcore_axis_name="s")
    n_chunks = b_per_w // _CHUNK
    n_sup = b_per_w // _SUP
    n_rounds = (n_sup + _NBUF - 1) // _NBUF

    @functools.partial(
        pl.kernel,
        mesh=mesh,
        out_type=jax.ShapeDtypeStruct((idx_flat.shape[0], _D), jnp.float32),
        scratch_types=[
            pltpu.VMEM((n_chunks, _CHUNK), jnp.int32),
            pltpu.VMEM((_NBUF, _SUP, _D), jnp.float32),
            pltpu.SemaphoreType.DMA((_NBUF,)),
            pltpu.SemaphoreType.DMA((_NBUF,)),
        ],
    )
    def k(idx_hbm, table_hbm, out_hbm, idx_v, rows_v, gsem, osem):
        num_cores = jax.lax.axis_size("c")
        wid = lax.axis_index("s") * num_cores + lax.axis_index("c")
        base = wid * b_per_w

        pltpu.sync_copy(idx_hbm.at[wid], idx_v)

        def gather_start(t, b):
            for g in range(_GPS):
                pltpu.async_copy(
                    table_hbm.at[idx_v.at[t * _GPS + g]],
                    rows_v.at[b].at[pl.ds(g * _CHUNK, _CHUNK)],
                    gsem.at[b],
                )

        def gather_wait(b):
            for g in range(_GPS):
                pltpu.make_async_copy(
                    table_hbm.at[idx_v.at[0]],
                    rows_v.at[b].at[pl.ds(g * _CHUNK, _CHUNK)],
                    gsem.at[b],
                ).wait()

        def out_start(t, b):
            pltpu.async_copy(
                rows_v.at[b], out_hbm.at[pl.ds(base + t * _SUP, _SUP)],
                osem.at[b],
            )

        def out_wait(b):
            pltpu.make_async_copy(
                rows_v.at[b], out_hbm.at[pl.ds(base, _SUP)], osem.at[b]
            ).wait()

        for t in range(_DIST):
            gather_start(t, t % _NBUF)

        def body(i, carry):
            for b in range(_NBUF):
                t = i * _NBUF + b
                bg = (b + _DIST) % _NBUF
                tg = t + _DIST

                @pl.when(jnp.logical_and(tg >= _NBUF, tg < n_sup))
                def _():
                    out_wait(bg)

                @pl.when(tg < n_sup)
                def _():
                    gather_start(tg, bg)

                @pl.when(t < n_sup)
                def _():
                    gather_wait(b)
                    out_start(t, b)

            return carry

        lax.fori_loop(0, n_rounds, body, 0)

        for b in range(_NBUF):
            out_wait(b)

    idx3 = idx_flat.reshape(num_workers, n_chunks, _CHUNK)
    return k(idx3, weight)


def kernel(token_ids, weight):
    n, t = token_ids.shape
    idx_flat = token_ids.T.reshape(-1).astype(jnp.int32)
    b = idx_flat.shape[0]
    num_workers = 32
    assert b % (num_workers * _SUP) == 0
    out = _embedding_lookup(
        idx_flat, weight, num_workers=num_workers, b_per_w=b // num_workers
    )
    return out.reshape(t, n, _D).transpose(1, 0, 2)
